# baseline (speedup 1.0000x reference)
import jax
import jax.numpy as jnp
from jax import lax
from jax.experimental import pallas as pl
from jax.experimental.pallas import tpu as pltpu
from jax.experimental.pallas import tpu_sc as plsc

EMB = 64
CAT = 2 * EMB
N = 4096 * 20
NC, NS = 2, 16
NW = NC * NS
NP = N // NW
C = 128
NCHUNK = NP // C
G = C // 16
PSTR = 17


def _sc_scores_body(user_hbm, itp_hbm, itn_hbm,
                    ucat_hbm, icat_hbm, q_hbm, b_hbm,
                    pi_hbm, ni_hbm, pp_hbm, np_hbm,
                    qp_hbm, bp_hbm, qn_hbm, bn_hbm,
                    uidx_v, pidx_v, nidx_v,
                    ua, pa, na, ub, pb, nb,
                    qpa, bpa, qna, bna, qpb, bpb, qnb, bnb,
                    s0, s1, s2, s3,
                    pad0, pad1, pad2, pad3,
                    sem0, sem1):
    wid = lax.axis_index("s") * NC + lax.axis_index("c")
    base = wid * NP
    lanes = lax.iota(jnp.int32, 16)
    lanes_pstr = lanes * PSTR
    rows = [[ua, pa, na], [ub, pb, nb]]
    qbs = [[qpa, bpa, qna, bna], [qpb, bpb, qnb, bnb]]
    scos = [s0, s1, s2, s3]
    pads = [pad0, pad1, pad2, pad3]
    sems = [sem0, sem1]

    pltpu.sync_copy(user_hbm.at[pl.ds(base, NP)], uidx_v)
    pltpu.sync_copy(itp_hbm.at[pl.ds(base, NP)], pidx_v)
    pltpu.sync_copy(itn_hbm.at[pl.ds(base, NP)], nidx_v)

    def copies(k, s):
        u = uidx_v.at[pl.ds(k * C, C)]
        p = pidx_v.at[pl.ds(k * C, C)]
        n = nidx_v.at[pl.ds(k * C, C)]
        R, Q = rows[s], qbs[s]
        return [
            pltpu.make_async_copy(ucat_hbm.at[u], R[0], sems[s]),
            pltpu.make_async_copy(icat_hbm.at[p], R[1], sems[s]),
            pltpu.make_async_copy(icat_hbm.at[n], R[2], sems[s]),
            pltpu.make_async_copy(q_hbm.at[p], Q[0], sems[s]),
            pltpu.make_async_copy(b_hbm.at[p], Q[1], sems[s]),
            pltpu.make_async_copy(q_hbm.at[n], Q[2], sems[s]),
            pltpu.make_async_copy(b_hbm.at[n], Q[3], sems[s]),
        ]

    for cp in copies(0, 0):
        cp.start()

    def compute(k, b):
        R, Q = rows[b], qbs[b]
        off = base + k * C

        def group(g, gcarry):
            row0 = g * 16
            for pp in range(16):
                p = row0 + pp
                xu = [R[0][p, pl.ds(16 * j, 16)] for j in range(4)]
                yu = [R[0][p, pl.ds(EMB + 16 * j, 16)] for j in range(4)]
                xp = [R[1][p, pl.ds(16 * j, 16)] for j in range(4)]
                yp = [R[1][p, pl.ds(EMB + 16 * j, 16)] for j in range(4)]
                xn = [R[2][p, pl.ds(16 * j, 16)] for j in range(4)]
                yn = [R[2][p, pl.ds(EMB + 16 * j, 16)] for j in range(4)]
                part = [
                    xu[0] * xp[0] + xu[1] * xp[1] + xu[2] * xp[2] + xu[3] * xp[3],
                    xu[0] * xn[0] + xu[1] * xn[1] + xu[2] * xn[2] + xu[3] * xn[3],
                    yu[0] * yp[0] + yu[1] * yp[1] + yu[2] * yp[2] + yu[3] * yp[3],
                    yu[0] * yn[0] + yu[1] * yn[1] + yu[2] * yn[2] + yu[3] * yn[3],
                ]
                for d in range(4):
                    pads[d][pl.ds(pp * PSTR, 16)] = part[d]
            for d in range(4):
                acc = plsc.load_gather(pads[d], [lanes_pstr])
                for c in range(1, 16):
                    acc = acc + plsc.load_gather(pads[d], [lanes_pstr + c])
                scos[d][pl.ds(row0, 16)] = acc
            return gcarry

        lax.fori_loop(0, G, group, 0)

        pltpu.sync_copy(scos[0], pi_hbm.at[pl.ds(off, C)])
        pltpu.sync_copy(scos[1], ni_hbm.at[pl.ds(off, C)])
        pltpu.sync_copy(scos[2], pp_hbm.at[pl.ds(off, C)])
        pltpu.sync_copy(scos[3], np_hbm.at[pl.ds(off, C)])
        pltpu.sync_copy(Q[0], qp_hbm.at[pl.ds(off, C)])
        pltpu.sync_copy(Q[1], bp_hbm.at[pl.ds(off, C)])
        pltpu.sync_copy(Q[2], qn_hbm.at[pl.ds(off, C)])
        pltpu.sync_copy(Q[3], bn_hbm.at[pl.ds(off, C)])

    def c2_body(c2, carry):
        for b in (0, 1):
            k = c2 * 2 + b
            for cp in copies(k, b):
                cp.wait()

            @pl.when(k + 1 < NCHUNK)
            def _prefetch():
                for cp in copies(k + 1, 1 - b):
                    cp.start()

            compute(k, b)
        return carry

    lax.fori_loop(0, NCHUNK // 2, c2_body, 0)


_f32n = jax.ShapeDtypeStruct((N,), jnp.float32)

_row_t = pltpu.VMEM((C, CAT), jnp.float32)
_c_t = pltpu.VMEM((C,), jnp.float32)
_pad_t = pltpu.VMEM((16 * PSTR,), jnp.float32)

_sc_scores = pl.kernel(
    _sc_scores_body,
    out_type=[_f32n] * 8,
    mesh=plsc.VectorSubcoreMesh(core_axis_name="c", subcore_axis_name="s"),
    compiler_params=pltpu.CompilerParams(needs_layout_passes=False),
    scratch_types=(
        [pltpu.VMEM((NP,), jnp.int32)] * 3
        + [_row_t] * 6
        + [_c_t] * 8
        + [_c_t] * 4
        + [_pad_t] * 4
        + [pltpu.SemaphoreType.DMA] * 2
    ),
)


def _loss_body(pi, ni, pp, np_, qp, bp, qn, bn, mk, out_ref):
    m = mk[...]
    p_int = pi[...]
    n_int = ni[...]
    p_pop = pp[...]
    n_pop = np_[...]

    def logsig(x):
        return jnp.log(jax.nn.sigmoid(x))

    inv_n = 1.0 / N
    loss_int = -jnp.sum(m * logsig(p_int - n_int)) * inv_n
    loss_pop = (-jnp.sum(m * logsig(n_pop - p_pop)) * inv_n
                - jnp.sum((1.0 - m) * logsig(p_pop - n_pop)) * inv_n)
    pop_p = jax.nn.softplus(qp[...]) + jax.nn.softplus(bp[...])
    pop_n = jax.nn.softplus(qn[...]) + jax.nn.softplus(bn[...])
    p_tdic = jnp.tanh(pop_p) * (p_int + p_pop)
    n_tdic = jnp.tanh(pop_n) * (n_int + n_pop)
    loss_tdic = -jnp.sum(logsig(p_tdic - n_tdic)) * inv_n
    out_ref[0, 0] = 0.5 * loss_int + 0.5 * loss_pop + 0.5 * loss_tdic


_tc_loss = pl.pallas_call(
    _loss_body,
    out_shape=jax.ShapeDtypeStruct((1, 1), jnp.float32),
    out_specs=pl.BlockSpec(memory_space=pltpu.SMEM),
)

_NROWS = 1000000


_SH = 4000
_NSH = _NROWS // _SH


def _concat_body(ui, up, ii, ip, ucat, icat,
                 vu0, vp0, vi0, vq0, vu1, vp1, vi1, vq1,
                 wu0, wi0, wu1, wi1,
                 sin0, sin1, sout0, sout1):
    vin = [[vu0, vp0, vi0, vq0], [vu1, vp1, vi1, vq1]]
    wout = [[wu0, wi0], [wu1, wi1]]
    sin = [sin0, sin1]
    sout = [sout0, sout1]

    def in_copies(s, b):
        sl = pl.ds(s * _SH, _SH)
        V = vin[b]
        return [
            pltpu.make_async_copy(ui.at[sl], V[0], sin[b]),
            pltpu.make_async_copy(up.at[sl], V[1], sin[b]),
            pltpu.make_async_copy(ii.at[sl], V[2], sin[b]),
            pltpu.make_async_copy(ip.at[sl], V[3], sin[b]),
        ]

    def out_copies(s, b):
        sl = pl.ds(s * _SH, _SH)
        W = wout[b]
        return [
            pltpu.make_async_copy(W[0], ucat.at[sl], sout[b]),
            pltpu.make_async_copy(W[1], icat.at[sl], sout[b]),
        ]

    for cp in in_copies(0, 0):
        cp.start()

    def s2_body(s2, carry):
        for b in (0, 1):
            s = s2 * 2 + b
            for cp in in_copies(s, b):
                cp.wait()

            @pl.when(s + 1 < _NSH)
            def _prefetch():
                for cp in in_copies(s + 1, 1 - b):
                    cp.start()

            @pl.when(s >= 2)
            def _drain_out():
                for cp in out_copies(s - 2, b):
                    cp.wait()

            V, W = vin[b], wout[b]
            W[0][:, 0:EMB] = V[0][...]
            W[0][:, EMB:CAT] = V[1][...]
            W[1][:, 0:EMB] = V[2][...]
            W[1][:, EMB:CAT] = V[3][...]
            for cp in out_copies(s, b):
                cp.start()
        return carry

    lax.fori_loop(0, _NSH // 2, s2_body, 0)
    for b in (0, 1):
        for cp in out_copies(_NSH - 2 + b, b):
            cp.wait()


_in_t = pltpu.VMEM((_SH, EMB), jnp.float32)
_out_t = pltpu.VMEM((_SH, CAT), jnp.float32)

_tc_concat = pl.pallas_call(
    _concat_body,
    in_specs=[pl.BlockSpec(memory_space=pl.ANY)] * 4,
    out_specs=[pl.BlockSpec(memory_space=pl.ANY)] * 2,
    out_shape=[jax.ShapeDtypeStruct((_NROWS, CAT), jnp.float32)] * 2,
    scratch_shapes=[_in_t] * 8 + [_out_t] * 4 + [pltpu.SemaphoreType.DMA] * 4,
)


def kernel(user, item_p, item_n, mask, users_int, users_pop,
           items_int, items_pop, q, b):
    uf = user.reshape(N)
    pf = item_p.reshape(N)
    nf = item_n.reshape(N)
    u_cat, i_cat = _tc_concat(users_int, users_pop, items_int, items_pop)
    pi, ni, pp, np_, qp, bp, qn, bn = _sc_scores(
        uf, pf, nf, u_cat, i_cat, q, b)
    shp = (N // 128, 128)
    maskf = mask.reshape(shp).astype(jnp.float32)
    loss = _tc_loss(pi.reshape(shp), ni.reshape(shp), pp.reshape(shp),
                    np_.reshape(shp), qp.reshape(shp), bp.reshape(shp),
                    qn.reshape(shp), bn.reshape(shp), maskf)
    return loss[0, 0]


# speedup vs baseline: 1.4198x; 1.4198x over previous
import jax
import jax.numpy as jnp
from jax import lax
from jax.experimental import pallas as pl
from jax.experimental.pallas import tpu as pltpu
from jax.experimental.pallas import tpu_sc as plsc

EMB = 64
CAT = 2 * EMB
N = 4096 * 20
NC, NS = 2, 16
NW = NC * NS
NP = N // NW
C = 128
NCHUNK = NP // C
G = C // 16
PSTR = 17


def _sc_scores_body(user_hbm, itp_hbm, itn_hbm,
                    ucat_hbm, icat_hbm, q_hbm, b_hbm,
                    pi_hbm, ni_hbm, pp_hbm, np_hbm,
                    qp_hbm, bp_hbm, qn_hbm, bn_hbm,
                    uidx_v, pidx_v, nidx_v,
                    ua, pa, na, ub, pb, nb,
                    qpa, bpa, qna, bna, qpb, bpb, qnb, bnb,
                    s0, s1, s2, s3,
                    pad0, pad1, pad2, pad3,
                    sem0, sem1):
    wid = lax.axis_index("s") * NC + lax.axis_index("c")
    base = wid * NP
    lanes = lax.iota(jnp.int32, 16)
    lanes_pstr = lanes * PSTR
    rows = [[ua, pa, na], [ub, pb, nb]]
    qbs = [[qpa, bpa, qna, bna], [qpb, bpb, qnb, bnb]]
    scos = [s0, s1, s2, s3]
    pads = [pad0, pad1, pad2, pad3]
    sems = [sem0, sem1]

    pltpu.sync_copy(user_hbm.at[pl.ds(base, NP)], uidx_v)
    pltpu.sync_copy(itp_hbm.at[pl.ds(base, NP)], pidx_v)
    pltpu.sync_copy(itn_hbm.at[pl.ds(base, NP)], nidx_v)

    def copies(k, s):
        u = uidx_v.at[pl.ds(k * C, C)]
        p = pidx_v.at[pl.ds(k * C, C)]
        n = nidx_v.at[pl.ds(k * C, C)]
        R, Q = rows[s], qbs[s]
        return [
            pltpu.make_async_copy(ucat_hbm.at[u], R[0], sems[s]),
            pltpu.make_async_copy(icat_hbm.at[p], R[1], sems[s]),
            pltpu.make_async_copy(icat_hbm.at[n], R[2], sems[s]),
            pltpu.make_async_copy(q_hbm.at[p], Q[0], sems[s]),
            pltpu.make_async_copy(b_hbm.at[p], Q[1], sems[s]),
            pltpu.make_async_copy(q_hbm.at[n], Q[2], sems[s]),
            pltpu.make_async_copy(b_hbm.at[n], Q[3], sems[s]),
        ]

    for cp in copies(0, 0):
        cp.start()

    def compute(k, b):
        R, Q = rows[b], qbs[b]
        off = base + k * C

        def group(g, gcarry):
            row0 = g * 16
            for pp in range(16):
                p = row0 + pp
                xu = [R[0][p, pl.ds(16 * j, 16)] for j in range(4)]
                yu = [R[0][p, pl.ds(EMB + 16 * j, 16)] for j in range(4)]
                xp = [R[1][p, pl.ds(16 * j, 16)] for j in range(4)]
                yp = [R[1][p, pl.ds(EMB + 16 * j, 16)] for j in range(4)]
                xn = [R[2][p, pl.ds(16 * j, 16)] for j in range(4)]
                yn = [R[2][p, pl.ds(EMB + 16 * j, 16)] for j in range(4)]
                part = [
                    xu[0] * xp[0] + xu[1] * xp[1] + xu[2] * xp[2] + xu[3] * xp[3],
                    xu[0] * xn[0] + xu[1] * xn[1] + xu[2] * xn[2] + xu[3] * xn[3],
                    yu[0] * yp[0] + yu[1] * yp[1] + yu[2] * yp[2] + yu[3] * yp[3],
                    yu[0] * yn[0] + yu[1] * yn[1] + yu[2] * yn[2] + yu[3] * yn[3],
                ]
                for d in range(4):
                    pads[d][pl.ds(pp * PSTR, 16)] = part[d]
            for d in range(4):
                acc = plsc.load_gather(pads[d], [lanes_pstr])
                for c in range(1, 16):
                    acc = acc + plsc.load_gather(pads[d], [lanes_pstr + c])
                scos[d][pl.ds(row0, 16)] = acc
            return gcarry

        lax.fori_loop(0, G, group, 0)

        pltpu.sync_copy(scos[0], pi_hbm.at[pl.ds(off, C)])
        pltpu.sync_copy(scos[1], ni_hbm.at[pl.ds(off, C)])
        pltpu.sync_copy(scos[2], pp_hbm.at[pl.ds(off, C)])
        pltpu.sync_copy(scos[3], np_hbm.at[pl.ds(off, C)])
        pltpu.sync_copy(Q[0], qp_hbm.at[pl.ds(off, C)])
        pltpu.sync_copy(Q[1], bp_hbm.at[pl.ds(off, C)])
        pltpu.sync_copy(Q[2], qn_hbm.at[pl.ds(off, C)])
        pltpu.sync_copy(Q[3], bn_hbm.at[pl.ds(off, C)])

    def c2_body(c2, carry):
        for b in (0, 1):
            k = c2 * 2 + b
            for cp in copies(k, b):
                cp.wait()

            @pl.when(k + 1 < NCHUNK)
            def _prefetch():
                for cp in copies(k + 1, 1 - b):
                    cp.start()

            compute(k, b)
        return carry

    lax.fori_loop(0, NCHUNK // 2, c2_body, 0)


_f32n = jax.ShapeDtypeStruct((N,), jnp.float32)

_row_t = pltpu.VMEM((C, CAT), jnp.float32)
_c_t = pltpu.VMEM((C,), jnp.float32)
_pad_t = pltpu.VMEM((16 * PSTR,), jnp.float32)

_sc_scores = pl.kernel(
    _sc_scores_body,
    out_type=[_f32n] * 8,
    mesh=plsc.VectorSubcoreMesh(core_axis_name="c", subcore_axis_name="s"),
    compiler_params=pltpu.CompilerParams(needs_layout_passes=False),
    scratch_types=(
        [pltpu.VMEM((NP,), jnp.int32)] * 3
        + [_row_t] * 6
        + [_c_t] * 8
        + [_c_t] * 4
        + [_pad_t] * 4
        + [pltpu.SemaphoreType.DMA] * 2
    ),
)


def _loss_body(pi, ni, pp, np_, qp, bp, qn, bn, mk, out_ref):
    m = mk[...]
    p_int = pi[...]
    n_int = ni[...]
    p_pop = pp[...]
    n_pop = np_[...]

    def logsig(x):
        return jnp.log(jax.nn.sigmoid(x))

    inv_n = 1.0 / N
    loss_int = -jnp.sum(m * logsig(p_int - n_int)) * inv_n
    loss_pop = (-jnp.sum(m * logsig(n_pop - p_pop)) * inv_n
                - jnp.sum((1.0 - m) * logsig(p_pop - n_pop)) * inv_n)
    pop_p = jax.nn.softplus(qp[...]) + jax.nn.softplus(bp[...])
    pop_n = jax.nn.softplus(qn[...]) + jax.nn.softplus(bn[...])
    p_tdic = jnp.tanh(pop_p) * (p_int + p_pop)
    n_tdic = jnp.tanh(pop_n) * (n_int + n_pop)
    loss_tdic = -jnp.sum(logsig(p_tdic - n_tdic)) * inv_n
    out_ref[0, 0] = 0.5 * loss_int + 0.5 * loss_pop + 0.5 * loss_tdic


_tc_loss = pl.pallas_call(
    _loss_body,
    out_shape=jax.ShapeDtypeStruct((1, 1), jnp.float32),
    out_specs=pl.BlockSpec(memory_space=pltpu.SMEM),
)


def kernel(user, item_p, item_n, mask, users_int, users_pop,
           items_int, items_pop, q, b):
    uf = user.reshape(N)
    pf = item_p.reshape(N)
    nf = item_n.reshape(N)
    u_cat = jnp.concatenate([users_int, users_pop], axis=1)
    i_cat = jnp.concatenate([items_int, items_pop], axis=1)
    pi, ni, pp, np_, qp, bp, qn, bn = _sc_scores(
        uf, pf, nf, u_cat, i_cat, q, b)
    shp = (N // 128, 128)
    maskf = mask.reshape(shp).astype(jnp.float32)
    loss = _tc_loss(pi.reshape(shp), ni.reshape(shp), pp.reshape(shp),
                    np_.reshape(shp), qp.reshape(shp), bp.reshape(shp),
                    qn.reshape(shp), bn.reshape(shp), maskf)
    return loss[0, 0]
